# baseline (speedup 1.0000x reference)
import jax
import jax.numpy as jnp
import numpy as np
from jax.experimental import pallas as pl
from jax.experimental.pallas import tpu as pltpu

B, S, D = 1, 2048, 1024
H, HD = 16, 64
E, K, F = 64, 2, 2048
EPS = 1e-6
THETA = 10000.0
NEG = -1e9

SBLK = 256
NI = S // SBLK
TT = 128
G = (S * K) // TT + E


def _rope(x, cos, sin):
    pieces = []
    for c in range(H):
        a = x[:, c * HD: c * HD + HD // 2]
        b = x[:, c * HD + HD // 2: (c + 1) * HD]
        pieces.append(-b)
        pieces.append(a)
    rot = jnp.concatenate(pieces, axis=1)
    return x * cos + rot * sin


def _qkv_kernel(x_ref, w_ref, lnw_ref, cos_ref, sin_ref, out_ref):
    x = x_ref[...]
    v = jnp.mean(x * x, axis=-1, keepdims=True)
    xn = x * jax.lax.rsqrt(v + EPS) * lnw_ref[...]
    y = jnp.dot(xn, w_ref[...], preferred_element_type=jnp.float32)
    cos = cos_ref[...]
    sin = sin_ref[...]
    q = _rope(y[:, :D], cos, sin)
    k = _rope(y[:, D:2 * D], cos, sin)
    out_ref[...] = jnp.concatenate([q, k, y[:, 2 * D:]], axis=1)


def _attn_kernel(q_ref, k_ref, v_ref, o_ref):
    i = pl.program_id(1)
    q = q_ref[0]
    s = jax.lax.dot_general(q, k_ref[0], (((1,), (1,)), ((), ())),
                            preferred_element_type=jnp.float32)
    s = s * (1.0 / np.sqrt(HD))
    rows = i * SBLK + jax.lax.broadcasted_iota(jnp.int32, (SBLK, S), 0)
    cols = jax.lax.broadcasted_iota(jnp.int32, (SBLK, S), 1)
    s = s + jnp.where(cols <= rows, 0.0, NEG)
    m = jnp.max(s, axis=1, keepdims=True)
    p = jnp.exp(s - m)
    p = p / jnp.sum(p, axis=1, keepdims=True)
    o_ref[0] = jnp.dot(p, v_ref[0], preferred_element_type=jnp.float32)


def _oproj_kernel(o_ref, res_ref, wo_ref, ln2_ref, wr_ref,
                  h_ref, hn_ref, lg_ref, i1_ref, i2_ref, w1_ref, w2_ref):
    h = jnp.dot(o_ref[...], wo_ref[...],
                preferred_element_type=jnp.float32) + res_ref[...]
    h_ref[...] = h
    v = jnp.mean(h * h, axis=-1, keepdims=True)
    hn = h * jax.lax.rsqrt(v + EPS) * ln2_ref[...]
    hn_ref[...] = hn
    lg = jnp.dot(hn, wr_ref[...], preferred_element_type=jnp.float32)
    lg_ref[...] = lg
    mx = jnp.max(lg, axis=1, keepdims=True)
    ex = jnp.exp(lg - mx)
    pr = ex / jnp.sum(ex, axis=1, keepdims=True)
    lane = jax.lax.broadcasted_iota(jnp.int32, (SBLK, E), 1)
    m1 = jnp.max(pr, axis=1, keepdims=True)
    i1 = jnp.min(jnp.where(pr == m1, lane, E), axis=1, keepdims=True)
    pr2 = jnp.where(lane == i1, -1.0, pr)
    m2 = jnp.max(pr2, axis=1, keepdims=True)
    i2 = jnp.min(jnp.where(pr2 == m2, lane, E), axis=1, keepdims=True)
    ssum = m1 + m2
    i1_ref[...] = i1
    i2_ref[...] = i2
    w1_ref[...] = m1 / ssum
    w2_ref[...] = m2 / ssum


def _moe_kernel(eid_ref, cnt_ref,
                hn_ref, res_ref, tokc_ref, tokr_ref, wt_ref,
                wg_ref, wu_ref, wd_ref,
                out_ref):
    t = pl.program_id(0)
    bf16 = jnp.bfloat16

    @pl.when(t == 0)
    def _init():
        out_ref[...] = res_ref[...]

    @pl.when(cnt_ref[t] > 0)
    def _body():
        tokc = tokc_ref[0]
        iot = jax.lax.broadcasted_iota(jnp.int32, (TT, S), 1)
        oh = (iot == tokc).astype(bf16)
        x = jnp.dot(oh, hn_ref[...],
                    preferred_element_type=jnp.float32).astype(bf16)
        g = jnp.dot(x, wg_ref[0], preferred_element_type=jnp.float32)
        u = jnp.dot(x, wu_ref[0], preferred_element_type=jnp.float32)
        p = (g * jax.lax.logistic(g) * u).astype(bf16)
        y = jnp.dot(p, wd_ref[0], preferred_element_type=jnp.float32)
        yw = (y * wt_ref[0]).astype(bf16)
        rowio = jax.lax.broadcasted_iota(jnp.int32, (S, TT), 0)
        oht = (rowio == tokr_ref[0]).astype(bf16)
        out_ref[...] += jnp.dot(oht, yw, preferred_element_type=jnp.float32)


def kernel(hidden_states, ln1_w, ln2_w, Wq, Wk, Wv, Wo, Wr, Wgate, Wup, Wdown):
    f32 = jnp.float32
    x = hidden_states.reshape(S, D)
    Wqkv = jnp.concatenate([Wq, Wk, Wv], axis=1)

    inv_freq = 1.0 / (THETA ** (jnp.arange(0, HD, 2, dtype=f32) / HD))
    t = jnp.arange(S, dtype=f32)
    freqs = jnp.outer(t, inv_freq)
    emb = jnp.concatenate([freqs, freqs], axis=-1)
    cos_t = jnp.tile(jnp.cos(emb), (1, H))
    sin_t = jnp.tile(jnp.sin(emb), (1, H))

    qkv = pl.pallas_call(
        _qkv_kernel,
        grid=(NI,),
        in_specs=[
            pl.BlockSpec((SBLK, D), lambda i: (i, 0)),
            pl.BlockSpec((D, 3 * D), lambda i: (0, 0)),
            pl.BlockSpec((1, D), lambda i: (0, 0)),
            pl.BlockSpec((SBLK, D), lambda i: (i, 0)),
            pl.BlockSpec((SBLK, D), lambda i: (i, 0)),
        ],
        out_specs=pl.BlockSpec((SBLK, 3 * D), lambda i: (i, 0)),
        out_shape=jax.ShapeDtypeStruct((S, 3 * D), f32),
    )(x, Wqkv, ln1_w.reshape(1, D), cos_t, sin_t)

    qkv3 = qkv.reshape(S, 3 * H, HD).transpose(1, 0, 2)

    o3 = pl.pallas_call(
        _attn_kernel,
        grid=(H, NI),
        in_specs=[
            pl.BlockSpec((1, SBLK, HD), lambda h, i: (h, i, 0)),
            pl.BlockSpec((1, S, HD), lambda h, i: (H + h, 0, 0)),
            pl.BlockSpec((1, S, HD), lambda h, i: (2 * H + h, 0, 0)),
        ],
        out_specs=pl.BlockSpec((1, SBLK, HD), lambda h, i: (h, i, 0)),
        out_shape=jax.ShapeDtypeStruct((H, S, HD), f32),
    )(qkv3, qkv3, qkv3)
    o = o3.transpose(1, 0, 2).reshape(S, D)

    h, hn, logits, i1, i2, w1, w2 = pl.pallas_call(
        _oproj_kernel,
        grid=(NI,),
        in_specs=[
            pl.BlockSpec((SBLK, D), lambda i: (i, 0)),
            pl.BlockSpec((SBLK, D), lambda i: (i, 0)),
            pl.BlockSpec((D, D), lambda i: (0, 0)),
            pl.BlockSpec((1, D), lambda i: (0, 0)),
            pl.BlockSpec((D, E), lambda i: (0, 0)),
        ],
        out_specs=[
            pl.BlockSpec((SBLK, D), lambda i: (i, 0)),
            pl.BlockSpec((SBLK, D), lambda i: (i, 0)),
            pl.BlockSpec((SBLK, E), lambda i: (i, 0)),
            pl.BlockSpec((SBLK, 1), lambda i: (i, 0)),
            pl.BlockSpec((SBLK, 1), lambda i: (i, 0)),
            pl.BlockSpec((SBLK, 1), lambda i: (i, 0)),
            pl.BlockSpec((SBLK, 1), lambda i: (i, 0)),
        ],
        out_shape=[
            jax.ShapeDtypeStruct((S, D), f32),
            jax.ShapeDtypeStruct((S, D), f32),
            jax.ShapeDtypeStruct((S, E), f32),
            jax.ShapeDtypeStruct((S, 1), jnp.int32),
            jax.ShapeDtypeStruct((S, 1), jnp.int32),
            jax.ShapeDtypeStruct((S, 1), f32),
            jax.ShapeDtypeStruct((S, 1), f32),
        ],
    )(o, x, Wo, ln2_w.reshape(1, D), Wr)

    experts = jnp.concatenate([i1[:, 0], i2[:, 0]])
    tokens = jnp.concatenate([jnp.arange(S, dtype=jnp.int32)] * 2)
    weights = jnp.concatenate([w1[:, 0], w2[:, 0]])
    order = jnp.argsort(experts)
    st = tokens[order]
    sw = weights[order]
    counts = jnp.zeros((E,), jnp.int32).at[experts].add(1)
    cum = jnp.cumsum(counts)
    offsets = cum - counts
    nt = (counts + TT - 1) // TT
    cumt = jnp.cumsum(nt)
    t_ar = jnp.arange(G, dtype=jnp.int32)
    e_of_t = jnp.searchsorted(cumt, t_ar, side='right').astype(jnp.int32)
    e_of_t = jnp.minimum(e_of_t, E - 1)
    local = t_ar - (cumt - nt)[e_of_t]
    cnt_t = jnp.clip(counts[e_of_t] - local * TT, 0, TT).astype(jnp.int32)
    eids = jax.lax.cummax(jnp.where(cnt_t > 0, e_of_t, 0))
    start = offsets[e_of_t] + local * TT
    idxs = start[:, None] + jnp.arange(TT, dtype=jnp.int32)[None]
    validm = jnp.arange(TT, dtype=jnp.int32)[None] < cnt_t[:, None]
    idxc = jnp.clip(idxs, 0, S * K - 1)
    tok_tab = jnp.where(validm, st[idxc], 0).astype(jnp.int32)
    w_tab = jnp.where(validm, sw[idxc], 0.0).astype(f32)

    tokc = tok_tab.reshape(G, TT, 1)
    tokr = tok_tab.reshape(G, 1, TT)
    wt = w_tab.reshape(G, TT, 1)

    bf16 = jnp.bfloat16
    out = pl.pallas_call(
        _moe_kernel,
        grid_spec=pltpu.PrefetchScalarGridSpec(
            num_scalar_prefetch=2,
            grid=(G,),
            in_specs=[
                pl.BlockSpec((S, D), lambda t, e, c: (0, 0)),
                pl.BlockSpec((S, D), lambda t, e, c: (0, 0)),
                pl.BlockSpec((1, TT, 1), lambda t, e, c: (t, 0, 0)),
                pl.BlockSpec((1, 1, TT), lambda t, e, c: (t, 0, 0)),
                pl.BlockSpec((1, TT, 1), lambda t, e, c: (t, 0, 0)),
                pl.BlockSpec((1, D, F), lambda t, e, c: (e[t], 0, 0)),
                pl.BlockSpec((1, D, F), lambda t, e, c: (e[t], 0, 0)),
                pl.BlockSpec((1, F, D), lambda t, e, c: (e[t], 0, 0)),
            ],
            out_specs=pl.BlockSpec((S, D), lambda t, e, c: (0, 0)),
        ),
        out_shape=jax.ShapeDtypeStruct((S, D), f32),
    )(eids, cnt_t, hn.astype(bf16), h, tokc, tokr, wt,
      Wgate.astype(bf16), Wup.astype(bf16), Wdown.astype(bf16))

    return (out.reshape(B, S, D), logits)


# speedup vs baseline: 1.3817x; 1.3817x over previous
import jax
import jax.numpy as jnp
import numpy as np
from jax.experimental import pallas as pl
from jax.experimental.pallas import tpu as pltpu

B, S, D = 1, 2048, 1024
H, HD = 16, 64
E, K, F = 64, 2, 2048
EPS = 1e-6
THETA = 10000.0
NEG = -1e9

SBLK = 256
NI = S // SBLK
TT = 128
G = (S * K) // TT + E
FB = 1024
NF = F // FB


def _rope(x, cos, sin):
    pieces = []
    for c in range(H):
        a = x[:, c * HD: c * HD + HD // 2]
        b = x[:, c * HD + HD // 2: (c + 1) * HD]
        pieces.append(-b)
        pieces.append(a)
    rot = jnp.concatenate(pieces, axis=1)
    return x * cos + rot * sin


def _qkv_kernel(x_ref, w_ref, lnw_ref, cos_ref, sin_ref, out_ref):
    x = x_ref[...]
    v = jnp.mean(x * x, axis=-1, keepdims=True)
    xn = x * jax.lax.rsqrt(v + EPS) * lnw_ref[...]
    y = jnp.dot(xn, w_ref[...], preferred_element_type=jnp.float32)
    cos = cos_ref[...]
    sin = sin_ref[...]
    q = _rope(y[:, :D], cos, sin)
    k = _rope(y[:, D:2 * D], cos, sin)
    out_ref[...] = jnp.concatenate([q, k, y[:, 2 * D:]], axis=1)


def _attn_kernel(q_ref, k_ref, v_ref, o_ref):
    i = pl.program_id(1)
    q = q_ref[0] * (1.0 / np.sqrt(HD))

    def step(j, carry):
        m, l, acc = carry
        kj = k_ref[0, pl.ds(j * SBLK, SBLK), :]
        s = jax.lax.dot_general(q, kj, (((1,), (1,)), ((), ())),
                                preferred_element_type=jnp.float32)
        rows = i * SBLK + jax.lax.broadcasted_iota(jnp.int32, (SBLK, SBLK), 0)
        cols = j * SBLK + jax.lax.broadcasted_iota(jnp.int32, (SBLK, SBLK), 1)
        s = s + jnp.where(cols <= rows, 0.0, NEG)
        mj = jnp.maximum(m, jnp.max(s, axis=1, keepdims=True))
        p = jnp.exp(s - mj)
        scale = jnp.exp(m - mj)
        vj = v_ref[0, pl.ds(j * SBLK, SBLK), :]
        acc = acc * scale + jnp.dot(p, vj, preferred_element_type=jnp.float32)
        l = l * scale[:, 0] + jnp.sum(p, axis=1)
        return mj, l, acc

    m0 = jnp.full((SBLK, 1), -jnp.inf, jnp.float32)
    l0 = jnp.zeros((SBLK,), jnp.float32)
    a0 = jnp.zeros((SBLK, HD), jnp.float32)
    m, l, acc = jax.lax.fori_loop(0, i + 1, step, (m0, l0, a0))
    o_ref[0] = acc / l[:, None]


def _oproj_kernel(o_ref, res_ref, wo_ref, ln2_ref, wr_ref,
                  h_ref, hn_ref, lg_ref, i1_ref, i2_ref, w1_ref, w2_ref):
    h = jnp.dot(o_ref[...], wo_ref[...],
                preferred_element_type=jnp.float32) + res_ref[...]
    h_ref[...] = h
    v = jnp.mean(h * h, axis=-1, keepdims=True)
    hn = h * jax.lax.rsqrt(v + EPS) * ln2_ref[...]
    hn_ref[...] = hn
    lg = jnp.dot(hn, wr_ref[...], preferred_element_type=jnp.float32)
    lg_ref[...] = lg
    mx = jnp.max(lg, axis=1, keepdims=True)
    ex = jnp.exp(lg - mx)
    pr = ex / jnp.sum(ex, axis=1, keepdims=True)
    lane = jax.lax.broadcasted_iota(jnp.int32, (SBLK, E), 1)
    m1 = jnp.max(pr, axis=1, keepdims=True)
    i1 = jnp.min(jnp.where(pr == m1, lane, E), axis=1, keepdims=True)
    pr2 = jnp.where(lane == i1, -1.0, pr)
    m2 = jnp.max(pr2, axis=1, keepdims=True)
    i2 = jnp.min(jnp.where(pr2 == m2, lane, E), axis=1, keepdims=True)
    ssum = m1 + m2
    i1_ref[...] = i1
    i2_ref[...] = i2
    w1_ref[...] = m1 / ssum
    w2_ref[...] = m2 / ssum


def _moe_kernel(eid_ref, cnt_ref,
                hn_ref, res_ref, tokc_ref, tokr_ref, wt_ref,
                wg_ref, wu_ref, wd_ref,
                out_ref, x_sc, y_sc):
    t = pl.program_id(0)
    f = pl.program_id(1)

    @pl.when((t == 0) & (f == 0))
    def _init():
        out_ref[...] = res_ref[...]

    @pl.when(cnt_ref[t] > 0)
    def _body():
        tokc = tokc_ref[0]

        @pl.when(f == 0)
        def _gather():
            iot = jax.lax.broadcasted_iota(jnp.int32, (TT, S), 1)
            oh = (iot == tokc).astype(jnp.float32)
            x_sc[...] = jnp.dot(oh, hn_ref[...],
                                preferred_element_type=jnp.float32)

        x = x_sc[...]
        g = jnp.dot(x, wg_ref[0], preferred_element_type=jnp.float32)
        u = jnp.dot(x, wu_ref[0], preferred_element_type=jnp.float32)
        p = g * jax.lax.logistic(g) * u
        yp = jnp.dot(p, wd_ref[0], preferred_element_type=jnp.float32)

        @pl.when(f == 0)
        def _acc0():
            y_sc[...] = yp

        @pl.when(f != 0)
        def _acc1():
            y_sc[...] += yp

        @pl.when(f == NF - 1)
        def _scatter():
            yw = y_sc[...] * wt_ref[0]
            rowio = jax.lax.broadcasted_iota(jnp.int32, (S, TT), 0)
            oht = (rowio == tokr_ref[0]).astype(jnp.float32)
            out_ref[...] += jnp.dot(oht, yw,
                                    preferred_element_type=jnp.float32)


def kernel(hidden_states, ln1_w, ln2_w, Wq, Wk, Wv, Wo, Wr, Wgate, Wup, Wdown):
    f32 = jnp.float32
    x = hidden_states.reshape(S, D)
    Wqkv = jnp.concatenate([Wq, Wk, Wv], axis=1)

    inv_freq = 1.0 / (THETA ** (jnp.arange(0, HD, 2, dtype=f32) / HD))
    t = jnp.arange(S, dtype=f32)
    freqs = jnp.outer(t, inv_freq)
    emb = jnp.concatenate([freqs, freqs], axis=-1)
    cos_t = jnp.tile(jnp.cos(emb), (1, H))
    sin_t = jnp.tile(jnp.sin(emb), (1, H))

    qkv = pl.pallas_call(
        _qkv_kernel,
        grid=(NI,),
        in_specs=[
            pl.BlockSpec((SBLK, D), lambda i: (i, 0)),
            pl.BlockSpec((D, 3 * D), lambda i: (0, 0)),
            pl.BlockSpec((1, D), lambda i: (0, 0)),
            pl.BlockSpec((SBLK, D), lambda i: (i, 0)),
            pl.BlockSpec((SBLK, D), lambda i: (i, 0)),
        ],
        out_specs=pl.BlockSpec((SBLK, 3 * D), lambda i: (i, 0)),
        out_shape=jax.ShapeDtypeStruct((S, 3 * D), f32),
    )(x, Wqkv, ln1_w.reshape(1, D), cos_t, sin_t)

    qkv3 = qkv.reshape(S, 3 * H, HD).transpose(1, 0, 2)

    o3 = pl.pallas_call(
        _attn_kernel,
        grid=(H, NI),
        in_specs=[
            pl.BlockSpec((1, SBLK, HD), lambda h, i: (h, i, 0)),
            pl.BlockSpec((1, S, HD), lambda h, i: (H + h, 0, 0)),
            pl.BlockSpec((1, S, HD), lambda h, i: (2 * H + h, 0, 0)),
        ],
        out_specs=pl.BlockSpec((1, SBLK, HD), lambda h, i: (h, i, 0)),
        out_shape=jax.ShapeDtypeStruct((H, S, HD), f32),
    )(qkv3, qkv3, qkv3)
    o = o3.transpose(1, 0, 2).reshape(S, D)

    h, hn, logits, i1, i2, w1, w2 = pl.pallas_call(
        _oproj_kernel,
        grid=(NI,),
        in_specs=[
            pl.BlockSpec((SBLK, D), lambda i: (i, 0)),
            pl.BlockSpec((SBLK, D), lambda i: (i, 0)),
            pl.BlockSpec((D, D), lambda i: (0, 0)),
            pl.BlockSpec((1, D), lambda i: (0, 0)),
            pl.BlockSpec((D, E), lambda i: (0, 0)),
        ],
        out_specs=[
            pl.BlockSpec((SBLK, D), lambda i: (i, 0)),
            pl.BlockSpec((SBLK, D), lambda i: (i, 0)),
            pl.BlockSpec((SBLK, E), lambda i: (i, 0)),
            pl.BlockSpec((SBLK, 1), lambda i: (i, 0)),
            pl.BlockSpec((SBLK, 1), lambda i: (i, 0)),
            pl.BlockSpec((SBLK, 1), lambda i: (i, 0)),
            pl.BlockSpec((SBLK, 1), lambda i: (i, 0)),
        ],
        out_shape=[
            jax.ShapeDtypeStruct((S, D), f32),
            jax.ShapeDtypeStruct((S, D), f32),
            jax.ShapeDtypeStruct((S, E), f32),
            jax.ShapeDtypeStruct((S, 1), jnp.int32),
            jax.ShapeDtypeStruct((S, 1), jnp.int32),
            jax.ShapeDtypeStruct((S, 1), f32),
            jax.ShapeDtypeStruct((S, 1), f32),
        ],
    )(o, x, Wo, ln2_w.reshape(1, D), Wr)

    experts = jnp.concatenate([i1[:, 0], i2[:, 0]])
    tokens = jnp.concatenate([jnp.arange(S, dtype=jnp.int32)] * 2)
    weights = jnp.concatenate([w1[:, 0], w2[:, 0]])
    order = jnp.argsort(experts)
    st = tokens[order]
    sw = weights[order]
    counts = jnp.zeros((E,), jnp.int32).at[experts].add(1)
    cum = jnp.cumsum(counts)
    offsets = cum - counts
    nt = (counts + TT - 1) // TT
    cumt = jnp.cumsum(nt)
    t_ar = jnp.arange(G, dtype=jnp.int32)
    e_of_t = jnp.searchsorted(cumt, t_ar, side='right').astype(jnp.int32)
    e_of_t = jnp.minimum(e_of_t, E - 1)
    local = t_ar - (cumt - nt)[e_of_t]
    cnt_t = jnp.clip(counts[e_of_t] - local * TT, 0, TT).astype(jnp.int32)
    eids = jax.lax.cummax(jnp.where(cnt_t > 0, e_of_t, 0))
    start = offsets[e_of_t] + local * TT
    idxs = start[:, None] + jnp.arange(TT, dtype=jnp.int32)[None]
    validm = jnp.arange(TT, dtype=jnp.int32)[None] < cnt_t[:, None]
    idxc = jnp.clip(idxs, 0, S * K - 1)
    tok_tab = jnp.where(validm, st[idxc], 0).astype(jnp.int32)
    w_tab = jnp.where(validm, sw[idxc], 0.0).astype(f32)

    tokc = tok_tab.reshape(G, TT, 1)
    tokr = tok_tab.reshape(G, 1, TT)
    wt = w_tab.reshape(G, TT, 1)

    out = pl.pallas_call(
        _moe_kernel,
        grid_spec=pltpu.PrefetchScalarGridSpec(
            num_scalar_prefetch=2,
            grid=(G, NF),
            in_specs=[
                pl.BlockSpec((S, D), lambda t, f, e, c: (0, 0)),
                pl.BlockSpec((S, D), lambda t, f, e, c: (0, 0)),
                pl.BlockSpec((1, TT, 1), lambda t, f, e, c: (t, 0, 0)),
                pl.BlockSpec((1, 1, TT), lambda t, f, e, c: (t, 0, 0)),
                pl.BlockSpec((1, TT, 1), lambda t, f, e, c: (t, 0, 0)),
                pl.BlockSpec((1, D, FB),
                             lambda t, f, e, c: (e[t], 0,
                                                 jnp.where(c[t] > 0, f, 0))),
                pl.BlockSpec((1, D, FB),
                             lambda t, f, e, c: (e[t], 0,
                                                 jnp.where(c[t] > 0, f, 0))),
                pl.BlockSpec((1, FB, D),
                             lambda t, f, e, c: (e[t],
                                                 jnp.where(c[t] > 0, f, 0), 0)),
            ],
            out_specs=pl.BlockSpec((S, D), lambda t, f, e, c: (0, 0)),
            scratch_shapes=[
                pltpu.VMEM((TT, D), f32),
                pltpu.VMEM((TT, D), f32),
            ],
        ),
        out_shape=jax.ShapeDtypeStruct((S, D), f32),
    )(eids, cnt_t, hn, h, tokc, tokr, wt, Wgate, Wup, Wdown)

    return (out.reshape(B, S, D), logits)


# speedup vs baseline: 1.6091x; 1.1646x over previous
import jax
import jax.numpy as jnp
import numpy as np
from jax.experimental import pallas as pl
from jax.experimental.pallas import tpu as pltpu

B, S, D = 1, 2048, 1024
H, HD = 16, 64
E, K, F = 64, 2, 2048
EPS = 1e-6
THETA = 10000.0
NEG = -1e9

SBLK = 256
NI = S // SBLK
TT = 128
G = (S * K) // TT + E
FB = 1024
NF = F // FB


def _rope(x, cos, sin):
    pieces = []
    for c in range(H):
        a = x[:, c * HD: c * HD + HD // 2]
        b = x[:, c * HD + HD // 2: (c + 1) * HD]
        pieces.append(-b)
        pieces.append(a)
    rot = jnp.concatenate(pieces, axis=1)
    return x * cos + rot * sin


def _qkv_kernel(x_ref, w_ref, lnw_ref, cos_ref, sin_ref, out_ref):
    x = x_ref[...]
    v = jnp.mean(x * x, axis=-1, keepdims=True)
    xn = x * jax.lax.rsqrt(v + EPS) * lnw_ref[...]
    y = jnp.dot(xn, w_ref[...], preferred_element_type=jnp.float32)
    cos = cos_ref[...]
    sin = sin_ref[...]
    q = _rope(y[:, :D], cos, sin)
    k = _rope(y[:, D:2 * D], cos, sin)
    out_ref[...] = jnp.concatenate([q, k, y[:, 2 * D:]], axis=1)


def _attn_kernel(q_ref, k_ref, v_ref, o_ref):
    i = pl.program_id(1)
    q = q_ref[0] * (1.0 / np.sqrt(HD))

    def step(j, carry):
        m, l, acc = carry
        kj = k_ref[0, pl.ds(j * SBLK, SBLK), :]
        s = jax.lax.dot_general(q, kj, (((1,), (1,)), ((), ())),
                                preferred_element_type=jnp.float32)
        rows = i * SBLK + jax.lax.broadcasted_iota(jnp.int32, (SBLK, SBLK), 0)
        cols = j * SBLK + jax.lax.broadcasted_iota(jnp.int32, (SBLK, SBLK), 1)
        s = s + jnp.where(cols <= rows, 0.0, NEG)
        mj = jnp.maximum(m, jnp.max(s, axis=1, keepdims=True))
        p = jnp.exp(s - mj)
        scale = jnp.exp(m - mj)
        vj = v_ref[0, pl.ds(j * SBLK, SBLK), :]
        acc = acc * scale + jnp.dot(p, vj, preferred_element_type=jnp.float32)
        l = l * scale[:, 0] + jnp.sum(p, axis=1)
        return mj, l, acc

    m0 = jnp.full((SBLK, 1), -jnp.inf, jnp.float32)
    l0 = jnp.zeros((SBLK,), jnp.float32)
    a0 = jnp.zeros((SBLK, HD), jnp.float32)
    m, l, acc = jax.lax.fori_loop(0, i + 1, step, (m0, l0, a0))
    o_ref[0] = acc / l[:, None]


def _oproj_kernel(o_ref, res_ref, wo_ref, ln2_ref, wr_ref,
                  h_ref, hn_ref, lg_ref, i1_ref, i2_ref, w1_ref, w2_ref):
    h = jnp.dot(o_ref[...], wo_ref[...],
                preferred_element_type=jnp.float32) + res_ref[...]
    h_ref[...] = h
    v = jnp.mean(h * h, axis=-1, keepdims=True)
    hn = h * jax.lax.rsqrt(v + EPS) * ln2_ref[...]
    hn_ref[...] = hn
    lg = jnp.dot(hn, wr_ref[...], preferred_element_type=jnp.float32)
    lg_ref[...] = lg
    mx = jnp.max(lg, axis=1, keepdims=True)
    ex = jnp.exp(lg - mx)
    pr = ex / jnp.sum(ex, axis=1, keepdims=True)
    lane = jax.lax.broadcasted_iota(jnp.int32, (SBLK, E), 1)
    m1 = jnp.max(pr, axis=1, keepdims=True)
    i1 = jnp.min(jnp.where(pr == m1, lane, E), axis=1, keepdims=True)
    pr2 = jnp.where(lane == i1, -1.0, pr)
    m2 = jnp.max(pr2, axis=1, keepdims=True)
    i2 = jnp.min(jnp.where(pr2 == m2, lane, E), axis=1, keepdims=True)
    ssum = m1 + m2
    i1_ref[...] = i1
    i2_ref[...] = i2
    w1_ref[...] = m1 / ssum
    w2_ref[...] = m2 / ssum


def _dispatch_kernel(i1_ref, i2_ref, ppos1_ref, ppos2_ref,
                     eid_ref, cnt_ref):
    bf16 = jnp.bfloat16
    f32 = jnp.float32
    lane = jax.lax.broadcasted_iota(jnp.int32, (S, E), 1)
    a1 = (lane == i1_ref[...]).astype(bf16)
    a2 = (lane == i2_ref[...]).astype(bf16)
    r_io = jax.lax.broadcasted_iota(jnp.int32, (S, S), 0)
    c_io = jax.lax.broadcasted_iota(jnp.int32, (S, S), 1)
    tl = (c_io < r_io).astype(bf16)
    p1 = jnp.dot(tl, a1, preferred_element_type=f32)
    p2 = jnp.dot(tl, a2, preferred_element_type=f32)
    a1f = a1.astype(f32)
    a2f = a2.astype(f32)
    c1 = jnp.sum(a1f, axis=0, keepdims=True)
    c2 = jnp.sum(a2f, axis=0, keepdims=True)
    counts = c1 + c2
    rank1 = jnp.sum(p1 * a1f, axis=1, keepdims=True)
    rank2 = jnp.sum((p2 + c1) * a2f, axis=1, keepdims=True)
    nt = jnp.floor((counts + (TT - 1)) * (1.0 / TT))
    elane = jax.lax.broadcasted_iota(jnp.int32, (E, E), 0)
    ecol = jax.lax.broadcasted_iota(jnp.int32, (E, E), 1)
    tu = (elane < ecol).astype(f32)
    tfirst = jnp.dot(nt, tu, preferred_element_type=f32)
    base1 = jnp.sum(a1f * (tfirst * TT), axis=1, keepdims=True)
    base2 = jnp.sum(a2f * (tfirst * TT), axis=1, keepdims=True)
    ppos1_ref[...] = (base1 + rank1).astype(jnp.int32)
    ppos2_ref[...] = (base2 + rank2).astype(jnp.int32)
    cum_incl = tfirst + nt
    tio = jax.lax.broadcasted_iota(jnp.int32, (G, E), 0).astype(f32)
    e_of_t = jnp.sum((cum_incl <= tio).astype(f32), axis=1, keepdims=True)
    e_of_t = jnp.minimum(e_of_t, float(E - 1))
    glane = jax.lax.broadcasted_iota(jnp.int32, (G, E), 1).astype(f32)
    sel_t = (glane == e_of_t).astype(f32)
    cnt_e = jnp.sum(sel_t * counts, axis=1, keepdims=True)
    tfirst_t = jnp.sum(sel_t * tfirst, axis=1, keepdims=True)
    t_col = jax.lax.broadcasted_iota(jnp.int32, (G, 1), 0).astype(f32)
    cnt_t = jnp.clip(cnt_e - (t_col - tfirst_t) * TT, 0.0, float(TT))
    eid_ref[...] = e_of_t.astype(jnp.int32)
    cnt_ref[...] = cnt_t.astype(jnp.int32)


def _moe_kernel(eid_ref, cnt_ref,
                hn_ref, res_ref, tokc_ref, tokr_ref, wt_ref,
                wg_ref, wu_ref, wd_ref,
                out_ref, x_sc, y_sc):
    t = pl.program_id(0)
    f = pl.program_id(1)

    @pl.when((t == 0) & (f == 0))
    def _init():
        out_ref[...] = res_ref[...]

    @pl.when(cnt_ref[t] > 0)
    def _body():
        tokc = tokc_ref[0]

        @pl.when(f == 0)
        def _gather():
            iot = jax.lax.broadcasted_iota(jnp.int32, (TT, S), 1)
            oh = (iot == tokc).astype(jnp.float32)
            x_sc[...] = jnp.dot(oh, hn_ref[...],
                                preferred_element_type=jnp.float32)

        x = x_sc[...]
        g = jnp.dot(x, wg_ref[0], preferred_element_type=jnp.float32)
        u = jnp.dot(x, wu_ref[0], preferred_element_type=jnp.float32)
        p = g * jax.lax.logistic(g) * u
        yp = jnp.dot(p, wd_ref[0], preferred_element_type=jnp.float32)

        @pl.when(f == 0)
        def _acc0():
            y_sc[...] = yp

        @pl.when(f != 0)
        def _acc1():
            y_sc[...] += yp

        @pl.when(f == NF - 1)
        def _scatter():
            yw = y_sc[...] * wt_ref[0]
            rowio = jax.lax.broadcasted_iota(jnp.int32, (S, TT), 0)
            oht = (rowio == tokr_ref[0]).astype(jnp.float32)
            out_ref[...] += jnp.dot(oht, yw,
                                    preferred_element_type=jnp.float32)


def kernel(hidden_states, ln1_w, ln2_w, Wq, Wk, Wv, Wo, Wr, Wgate, Wup, Wdown):
    f32 = jnp.float32
    x = hidden_states.reshape(S, D)
    Wqkv = jnp.concatenate([Wq, Wk, Wv], axis=1)

    inv_freq = 1.0 / (THETA ** (jnp.arange(0, HD, 2, dtype=f32) / HD))
    t = jnp.arange(S, dtype=f32)
    freqs = jnp.outer(t, inv_freq)
    emb = jnp.concatenate([freqs, freqs], axis=-1)
    cos_t = jnp.tile(jnp.cos(emb), (1, H))
    sin_t = jnp.tile(jnp.sin(emb), (1, H))

    qkv = pl.pallas_call(
        _qkv_kernel,
        grid=(NI,),
        in_specs=[
            pl.BlockSpec((SBLK, D), lambda i: (i, 0)),
            pl.BlockSpec((D, 3 * D), lambda i: (0, 0)),
            pl.BlockSpec((1, D), lambda i: (0, 0)),
            pl.BlockSpec((SBLK, D), lambda i: (i, 0)),
            pl.BlockSpec((SBLK, D), lambda i: (i, 0)),
        ],
        out_specs=pl.BlockSpec((SBLK, 3 * D), lambda i: (i, 0)),
        out_shape=jax.ShapeDtypeStruct((S, 3 * D), f32),
    )(x, Wqkv, ln1_w.reshape(1, D), cos_t, sin_t)

    qkv3 = qkv.reshape(S, 3 * H, HD).transpose(1, 0, 2)

    o3 = pl.pallas_call(
        _attn_kernel,
        grid=(H, NI),
        in_specs=[
            pl.BlockSpec((1, SBLK, HD), lambda h, i: (h, i, 0)),
            pl.BlockSpec((1, S, HD), lambda h, i: (H + h, 0, 0)),
            pl.BlockSpec((1, S, HD), lambda h, i: (2 * H + h, 0, 0)),
        ],
        out_specs=pl.BlockSpec((1, SBLK, HD), lambda h, i: (h, i, 0)),
        out_shape=jax.ShapeDtypeStruct((H, S, HD), f32),
    )(qkv3, qkv3, qkv3)
    o = o3.transpose(1, 0, 2).reshape(S, D)

    h, hn, logits, i1, i2, w1, w2 = pl.pallas_call(
        _oproj_kernel,
        grid=(NI,),
        in_specs=[
            pl.BlockSpec((SBLK, D), lambda i: (i, 0)),
            pl.BlockSpec((SBLK, D), lambda i: (i, 0)),
            pl.BlockSpec((D, D), lambda i: (0, 0)),
            pl.BlockSpec((1, D), lambda i: (0, 0)),
            pl.BlockSpec((D, E), lambda i: (0, 0)),
        ],
        out_specs=[
            pl.BlockSpec((SBLK, D), lambda i: (i, 0)),
            pl.BlockSpec((SBLK, D), lambda i: (i, 0)),
            pl.BlockSpec((SBLK, E), lambda i: (i, 0)),
            pl.BlockSpec((SBLK, 1), lambda i: (i, 0)),
            pl.BlockSpec((SBLK, 1), lambda i: (i, 0)),
            pl.BlockSpec((SBLK, 1), lambda i: (i, 0)),
            pl.BlockSpec((SBLK, 1), lambda i: (i, 0)),
        ],
        out_shape=[
            jax.ShapeDtypeStruct((S, D), f32),
            jax.ShapeDtypeStruct((S, D), f32),
            jax.ShapeDtypeStruct((S, E), f32),
            jax.ShapeDtypeStruct((S, 1), jnp.int32),
            jax.ShapeDtypeStruct((S, 1), jnp.int32),
            jax.ShapeDtypeStruct((S, 1), f32),
            jax.ShapeDtypeStruct((S, 1), f32),
        ],
    )(o, x, Wo, ln2_w.reshape(1, D), Wr)

    ppos1, ppos2, eids2, cnts2 = pl.pallas_call(
        _dispatch_kernel,
        in_specs=[
            pl.BlockSpec((S, 1), lambda: (0, 0)),
            pl.BlockSpec((S, 1), lambda: (0, 0)),
        ],
        out_specs=[
            pl.BlockSpec((S, 1), lambda: (0, 0)),
            pl.BlockSpec((S, 1), lambda: (0, 0)),
            pl.BlockSpec((G, 1), lambda: (0, 0)),
            pl.BlockSpec((G, 1), lambda: (0, 0)),
        ],
        out_shape=[
            jax.ShapeDtypeStruct((S, 1), jnp.int32),
            jax.ShapeDtypeStruct((S, 1), jnp.int32),
            jax.ShapeDtypeStruct((G, 1), jnp.int32),
            jax.ShapeDtypeStruct((G, 1), jnp.int32),
        ],
    )(i1, i2)
    eids = eids2[:, 0]
    cnt_t = cnts2[:, 0]
    ppos = jnp.concatenate([ppos1[:, 0], ppos2[:, 0]])
    toks = jnp.concatenate([jnp.arange(S, dtype=jnp.int32)] * 2)
    wvals = jnp.concatenate([w1[:, 0], w2[:, 0]])
    tok_tab = jnp.zeros((G * TT,), jnp.int32).at[ppos].set(toks)
    w_tab = jnp.zeros((G * TT,), f32).at[ppos].set(wvals)

    tokc = tok_tab.reshape(G, TT, 1)
    tokr = tok_tab.reshape(G, 1, TT)
    wt = w_tab.reshape(G, TT, 1)

    out = pl.pallas_call(
        _moe_kernel,
        grid_spec=pltpu.PrefetchScalarGridSpec(
            num_scalar_prefetch=2,
            grid=(G, NF),
            in_specs=[
                pl.BlockSpec((S, D), lambda t, f, e, c: (0, 0)),
                pl.BlockSpec((S, D), lambda t, f, e, c: (0, 0)),
                pl.BlockSpec((1, TT, 1), lambda t, f, e, c: (t, 0, 0)),
                pl.BlockSpec((1, 1, TT), lambda t, f, e, c: (t, 0, 0)),
                pl.BlockSpec((1, TT, 1), lambda t, f, e, c: (t, 0, 0)),
                pl.BlockSpec((1, D, FB),
                             lambda t, f, e, c: (e[t], 0,
                                                 jnp.where(c[t] > 0, f, 0))),
                pl.BlockSpec((1, D, FB),
                             lambda t, f, e, c: (e[t], 0,
                                                 jnp.where(c[t] > 0, f, 0))),
                pl.BlockSpec((1, FB, D),
                             lambda t, f, e, c: (e[t],
                                                 jnp.where(c[t] > 0, f, 0), 0)),
            ],
            out_specs=pl.BlockSpec((S, D), lambda t, f, e, c: (0, 0)),
            scratch_shapes=[
                pltpu.VMEM((TT, D), f32),
                pltpu.VMEM((TT, D), f32),
            ],
        ),
        out_shape=jax.ShapeDtypeStruct((S, D), f32),
    )(eids, cnt_t, hn, h, tokc, tokr, wt, Wgate, Wup, Wdown)

    return (out.reshape(B, S, D), logits)
